# Initial kernel scaffold; baseline (speedup 1.0000x reference)
import jax
import jax.numpy as jnp
from jax.experimental import pallas as pl


def kernel(x, edge_index, W_emb, b_emb, W_conv, b_conv, W_c1, b_c1, W_c2, b_c2):
    raise NotImplementedError("write your pallas kernel here")


# speedup vs baseline: 21.5003x; 21.5003x over previous
import functools

import jax
import jax.numpy as jnp
from jax import lax
from jax.experimental import pallas as pl
from jax.experimental.pallas import tpu as pltpu
from jax.experimental.pallas import tpu_sc as plsc

N_NODES = 10000
N_EDGES = 160000
TOP_N = 128
CFG = 512
B = 4
HH = 64
WW = 64
HWFLAT = HH * WW

NPAD = 10240
KCOLS = 640
NC, NS = 2, 16
NWORK = NC * NS
EPAD = 163840
ROWS_A = 40
ROWS_C = 80
RNG_ROWS = 1280
NRANGE = NPAD // RNG_ROWS // NC
RNG_ELEMS = RNG_ROWS * KCOLS
CNT_DUMP = RNG_ELEMS
STRIPE = RNG_ELEMS // NS
ZCHUNK = 12800

def _edge_stats_body(dst_hbm, src_hbm, deg_out, m0_out,
                     dstv, srcv, updv, onesv, zerov, deg_s, m0_s):
    c = lax.axis_index("c")
    s = lax.axis_index("s")
    wid = c * NS + s

    pltpu.sync_copy(dst_hbm.at[wid], dstv)
    pltpu.sync_copy(src_hbm.at[wid], srcv)

    for t in range(8):
        onesv[pl.ds(16 * t, 16)] = jnp.ones((16,), jnp.float32)

    @pl.loop(0, (NPAD // NS) // 16)
    def _zinit(i):
        zerov[pl.ds(i * 16, 16)] = jnp.zeros((16,), jnp.float32)

    stripe = NPAD // NS
    pltpu.sync_copy(zerov, deg_s.at[pl.ds(s * stripe, stripe)])
    pltpu.sync_copy(zerov, m0_s.at[pl.ds(s * stripe, stripe)])
    plsc.subcore_barrier()

    @pl.loop(0, ROWS_A)
    def _scatter(j):
        for t in range(8):
            d16 = dstv[j, pl.ds(16 * t, 16)]
            upd = jnp.where(d16 == 0, 1.0, 0.0).astype(jnp.float32)
            updv[j, pl.ds(16 * t, 16)] = upd
        pltpu.sync_copy(onesv, deg_s.at[dstv.at[j]], add=True)
        pltpu.sync_copy(updv.at[j], m0_s.at[srcv.at[j]], add=True)

    plsc.subcore_barrier()
    pltpu.sync_copy(deg_s.at[pl.ds(s * stripe, stripe)],
                    deg_out.at[c, pl.ds(s * stripe, stripe)])
    pltpu.sync_copy(m0_s.at[pl.ds(s * stripe, stripe)],
                    m0_out.at[c, pl.ds(s * stripe, stripe)])


def _cnt_build_body(src_hbm, dst_hbm, rank_hbm, cnt_out,
                    srcv, dstv, offv, rankrow, onesv, zerov, rank_sp, cnt_s):
    c = lax.axis_index("c")
    s = lax.axis_index("s")

    pltpu.sync_copy(src_hbm.at[s], srcv)
    pltpu.sync_copy(dst_hbm.at[s], dstv)

    @pl.when(s == 0)
    def _stage_rank():
        pltpu.sync_copy(rank_hbm, rank_sp)

    for t in range(8):
        onesv[pl.ds(16 * t, 16)] = jnp.ones((16,), jnp.float32)

    @pl.loop(0, ZCHUNK // 16)
    def _zinit(i):
        zerov[pl.ds(i * 16, 16)] = jnp.zeros((16,), jnp.float32)

    plsc.subcore_barrier()

    @pl.loop(0, ROWS_C)
    def _gather(j):
        pltpu.sync_copy(rank_sp.at[srcv.at[j]], rankrow.at[j])

    lane = lax.iota(jnp.int32, 16)

    for r in range(NRANGE):
        base = (NRANGE * c + r) * RNG_ROWS

        for z in range(STRIPE // ZCHUNK):
            pltpu.sync_copy(
                zerov, cnt_s.at[pl.ds(s * STRIPE + z * ZCHUNK, ZCHUNK)])

        @pl.when(s == 0)
        def _zdump():
            pltpu.sync_copy(zerov.at[pl.ds(0, 128)],
                            cnt_s.at[pl.ds(CNT_DUMP, 128)])

        plsc.subcore_barrier()

        @pl.loop(0, ROWS_C)
        def _scatter(j):
            for t in range(8):
                d16 = dstv[j, pl.ds(16 * t, 16)]
                r16 = rankrow[j, pl.ds(16 * t, 16)]
                rel = d16 - base
                ok = (rel >= 0) & (rel < RNG_ROWS)
                off = jnp.where(ok, rel * KCOLS + r16,
                                CNT_DUMP + 16 * t + lane)
                offv[j, pl.ds(16 * t, 16)] = off
            pltpu.sync_copy(onesv, cnt_s.at[offv.at[j]], add=True)

        plsc.subcore_barrier()
        pltpu.sync_copy(
            cnt_s.at[pl.ds(s * STRIPE, STRIPE)],
            cnt_out.at[pl.ds(base * KCOLS + s * STRIPE, STRIPE)])
        plsc.subcore_barrier()


@functools.lru_cache(maxsize=None)
def _sc_kernels():
    mesh = plsc.VectorSubcoreMesh(core_axis_name="c", subcore_axis_name="s",
                                  num_cores=NC, num_subcores=NS)
    edge_stats = functools.partial(
        pl.kernel,
        out_type=(
            jax.ShapeDtypeStruct((NC, NPAD), jnp.float32),
            jax.ShapeDtypeStruct((NC, NPAD), jnp.float32),
        ),
        mesh=mesh,
        scratch_types=[
            pltpu.VMEM((ROWS_A, 128), jnp.int32),
            pltpu.VMEM((ROWS_A, 128), jnp.int32),
            pltpu.VMEM((ROWS_A, 128), jnp.float32),
            pltpu.VMEM((128,), jnp.float32),
            pltpu.VMEM((NPAD // NS,), jnp.float32),
            pltpu.VMEM_SHARED((NPAD,), jnp.float32),
            pltpu.VMEM_SHARED((NPAD,), jnp.float32),
        ],
    )(_edge_stats_body)
    cnt_build = functools.partial(
        pl.kernel,
        out_type=jax.ShapeDtypeStruct((NPAD * KCOLS,), jnp.float32),
        mesh=mesh,
        scratch_types=[
            pltpu.VMEM((ROWS_C, 128), jnp.int32),
            pltpu.VMEM((ROWS_C, 128), jnp.int32),
            pltpu.VMEM((ROWS_C, 128), jnp.int32),
            pltpu.VMEM((ROWS_C, 128), jnp.int32),
            pltpu.VMEM((128,), jnp.float32),
            pltpu.VMEM((ZCHUNK,), jnp.float32),
            pltpu.VMEM_SHARED((NPAD,), jnp.int32),
            pltpu.VMEM_SHARED((RNG_ELEMS + 128,), jnp.float32),
        ],
    )(_cnt_build_body)
    return edge_stats, cnt_build


def _embed_body(x_ref, wemb_ref, bemb_ref, wconv_ref, bconv_ref, dv_ref, s_ref):
    xf = x_ref[...]
    col = lax.broadcasted_iota(jnp.int32, (B, HWFLAT), 1)
    selcol = lax.broadcasted_iota(jnp.int32, (B, TOP_N), 1)

    def pick(j, carry):
        xc, vals, rows, cols = carry
        m = jnp.max(xc, axis=1, keepdims=True)
        hit = xc == m
        idx = jnp.min(jnp.where(hit, col, HWFLAT), axis=1, keepdims=True)
        xc = jnp.where(col == idx, -jnp.inf, xc)
        sel = selcol == j
        vals = jnp.where(sel, m, vals)
        rows = jnp.where(sel, (idx // WW).astype(jnp.float32) / 3.0, rows)
        cols = jnp.where(sel, (idx % WW).astype(jnp.float32) / 3.0, cols)
        return xc, vals, rows, cols

    zz = jnp.zeros((B, TOP_N), jnp.float32)
    _, vals, rows, cols = lax.fori_loop(0, TOP_N, pick, (xf, zz, zz, zz))
    feat = jnp.concatenate([vals, rows, cols], axis=1)
    e = jnp.maximum(
        jnp.dot(feat, wemb_ref[...], precision="highest") + bemb_ref[...], 0.0)
    u = jnp.dot(e, wconv_ref[...], precision="highest")
    dv = dv_ref[...]
    for i in range(B):
        ri = jnp.maximum(dv * u[i:i + 1, :] + bconv_ref[...], 0.0)
        s_ref[i] = jnp.dot(ri, wconv_ref[...], precision="highest")


def _embed(xf, w_emb, b_emb, w_conv, b_conv, dvals, interpret=False):
    return pl.pallas_call(
        _embed_body,
        out_shape=jax.ShapeDtypeStruct((B, KCOLS, CFG), jnp.float32),
        interpret=interpret,
    )(xf, w_emb, b_emb, w_conv, b_conv, dvals)


def _gcn_body(cnt_ref, m0_ref, s_ref, bconv_ref, wconv_ref,
              wc1_ref, bc1_ref, wc2t_ref, out_ref, vacc):
    i = pl.program_id(0)

    @pl.when(i == 0)
    def _init():
        vacc[...] = jnp.zeros((B, CFG), jnp.float32)

    cnt = cnt_ref[...]
    m0 = m0_ref[0]
    for b in range(B):
        q = jnp.dot(cnt, s_ref[b], precision="highest")
        h2 = jnp.maximum(q + bconv_ref[...], 0.0)
        vacc[pl.ds(b, 1), :] += jnp.dot(m0, h2, precision="highest")

    @pl.when(i == pl.num_programs(0) - 1)
    def _fin():
        v = vacc[...]
        h3 = jnp.maximum(
            jnp.dot(v, wconv_ref[...], precision="highest") + bconv_ref[...],
            0.0)
        h4 = jnp.maximum(
            jnp.dot(h3, wc1_ref[...], precision="highest") + bc1_ref[...], 0.0)
        o = jnp.sum(h4 * wc2t_ref[...], axis=1, keepdims=True)
        out_ref[...] = jnp.broadcast_to(o, (B, 128))


def _gcn_head(cnt, m0r, s_all, b_conv, w_conv, w_c1, b_c1, w_c2t,
              interpret=False):
    nblk = NPAD // 1024
    return pl.pallas_call(
        _gcn_body,
        grid=(nblk,),
        in_specs=[
            pl.BlockSpec((1024, KCOLS), lambda i: (i, 0)),
            pl.BlockSpec((1, 1, 1024), lambda i: (i, 0, 0)),
            pl.BlockSpec((B, KCOLS, CFG), lambda i: (0, 0, 0)),
            pl.BlockSpec((1, CFG), lambda i: (0, 0)),
            pl.BlockSpec((CFG, CFG), lambda i: (0, 0)),
            pl.BlockSpec((CFG, CFG), lambda i: (0, 0)),
            pl.BlockSpec((1, CFG), lambda i: (0, 0)),
            pl.BlockSpec((1, CFG), lambda i: (0, 0)),
        ],
        out_specs=pl.BlockSpec((B, 128), lambda i: (0, 0)),
        out_shape=jax.ShapeDtypeStruct((B, 128), jnp.float32),
        scratch_shapes=[pltpu.VMEM((B, CFG), jnp.float32)],
        interpret=interpret,
    )(cnt, m0r, s_all, b_conv, w_conv, w_c1, b_c1, w_c2t)


def kernel(x, edge_index, W_emb, b_emb, W_conv, b_conv, W_c1, b_c1, W_c2, b_c2):
    src = edge_index[0].astype(jnp.int32)
    dst = edge_index[1].astype(jnp.int32)
    pad = N_NODES + (jnp.arange(EPAD - N_EDGES, dtype=jnp.int32) % (NPAD - N_NODES))
    src_p = jnp.concatenate([src, pad])
    dst_p = jnp.concatenate([dst, pad])
    src_a = src_p.reshape(NWORK, ROWS_A, 128)
    dst_a = dst_p.reshape(NWORK, ROWS_A, 128)
    src_c = src_p.reshape(NS, ROWS_C, 128)
    dst_c = dst_p.reshape(NS, ROWS_C, 128)

    _edge_stats, _cnt_build = _sc_kernels()
    deg_p, m0_p = _edge_stats(dst_a, src_a)
    deg = (deg_p[0] + deg_p[1])[:N_NODES]
    m0 = m0_p[0] + m0_p[1]

    dvals = jnp.unique(deg, size=KCOLS - 1, fill_value=jnp.float32(2e9))
    rank = jnp.searchsorted(dvals, deg).astype(jnp.int32)
    rank_full = jnp.concatenate(
        [rank, jnp.full((NPAD - N_NODES,), KCOLS - 1, jnp.int32)])
    dv_full = jnp.concatenate(
        [dvals, jnp.float32(2e9)[None]]).reshape(KCOLS, 1)

    cnt = _cnt_build(src_c, dst_c, rank_full).reshape(NPAD, KCOLS)
    s_all = _embed(x.reshape(B, HWFLAT), W_emb, b_emb.reshape(1, CFG),
                   W_conv, b_conv.reshape(1, CFG), dv_full)

    outp = _gcn_head(cnt, m0.reshape(NPAD // 1024, 1, 1024), s_all,
                     b_conv.reshape(1, CFG), W_conv, W_c1,
                     b_c1.reshape(1, CFG), W_c2.reshape(1, CFG))
    return outp[:, :1] + b_c2[None, :]
